# baseline (device time: 512608 ns/iter reference)
import jax
import jax.numpy as jnp
from jax import lax
from jax.experimental import pallas as pl
from jax.experimental.pallas import tpu as pltpu

M = 4096
N = 4096
K = 8192
BLK = M // 8

AXIS_W = {"x": 4, "y": 2, "z": 1}

STRIPES = (
    (0,    1024, ("z", "x", "y")),
    (1024, 1024, ("x", "y", "z")),
    (2048,  512, ("z", "x", "y")),
    (2560,  512, ("y", "x", "z")),
    (3072,  512, ("x", "y", "z")),
    (3584,  512, ("x", "z", "y")),
)
N_STRIPES = len(STRIPES)
WMAX = 1024
KT = 2048
NKT = K // KT
TOT = N_STRIPES * NKT

def _stage_slot(s, j):
    return (4 * s + 6, 4 * s + 10, 4 * s + 16)[j]

N_AG = 34


def _runs(exchanged):
    offs = [0]
    for a in exchanged:
        w = AXIS_W[a]
        offs = sorted(o + d for o in offs for d in (0, w))
    runs = []
    start, length = offs[0], 1
    for o in offs[1:]:
        if o == start + length:
            length += 1
        else:
            runs.append((start, length))
            start, length = o, 1
    runs.append((start, length))
    return runs


def _fused_body(dy_ref, w_ref, out_ref, a_buf, b_buf, acc, rs_buf,
                a_sems, b_sems, rs_send, rs_recv, ag_send, ag_recv, cp_sems):
    x = lax.axis_index("x")
    y = lax.axis_index("y")
    z = lax.axis_index("z")
    idx = {"x": x, "y": y, "z": z}
    m = 2 * y + z
    b_own = 4 * x + m
    own_rows = pl.ds(b_own * BLK, BLK)

    def nbr_of(a):
        return tuple(1 - idx[ax] if ax == a else idx[ax] for ax in "xyz")

    def start_loads(g):
        s, kt = divmod(g, NKT)
        c0, w, _ = STRIPES[s]
        pa = g % 2
        descs = []
        for i, row0 in enumerate((m * BLK, (m + 4) * BLK)):
            d = pltpu.make_async_copy(
                dy_ref.at[pl.ds(row0, BLK), pl.ds(kt * KT, KT)],
                a_buf.at[pa, pl.ds(i * BLK, BLK), :],
                a_sems.at[pa, i],
            )
            d.start()
            descs.append(d)
        d = pltpu.make_async_copy(
            w_ref.at[pl.ds(c0, w), pl.ds(kt * KT, KT)],
            b_buf.at[pa, pl.ds(0, w), :],
            b_sems.at[pa],
        )
        d.start()
        descs.append(d)
        return descs

    ag_i = [0]
    exchanged = {}
    hop_descs = {}
    rs_descs = {}

    cp_descs = {}

    def start_hop(s):
        c0, w, order = STRIPES[s]
        ex = exchanged[s]
        a = order[len(ex)]
        base = sum(AXIS_W[ax] * idx[ax] for ax in "xyz" if ax not in ex)
        cols = pl.ds(c0, w)
        descs = []
        for off, length in _runs(ex):
            rows = pl.ds((base + off) * BLK, length * BLK)
            src = rs_buf.at[s, :, pl.ds(0, w)] if not ex else out_ref.at[rows, cols]
            d = pltpu.make_async_remote_copy(
                src_ref=src,
                dst_ref=out_ref.at[rows, cols],
                send_sem=ag_send.at[ag_i[0]],
                recv_sem=ag_recv.at[ag_i[0]],
                device_id=nbr_of(a),
                device_id_type=pl.DeviceIdType.MESH,
            )
            d.start()
            descs.append(d)
            ag_i[0] += 1
        exchanged[s] = ex + (a,)
        hop_descs[s] = descs

    def stage(s, j):
        c0, w, _ = STRIPES[s]
        if j == 0:
            rs_descs[s].wait()
            sa = s % 2
            rs_buf[s, :, pl.ds(0, w)] = (
                rs_buf[s, :, pl.ds(0, w)]
                + acc[sa, pl.ds(x * BLK, BLK), pl.ds(0, w)]
            )
            cp = pltpu.make_async_copy(
                rs_buf.at[s, :, pl.ds(0, w)],
                out_ref.at[own_rows, pl.ds(c0, w)],
                cp_sems.at[s],
            )
            cp.start()
            cp_descs[s] = cp
            exchanged[s] = ()
            start_hop(s)
        else:
            if j == 1:
                cp_descs[s].wait()
            for d in hop_descs[s]:
                d.wait()
            start_hop(s)

    slot_stages = {}
    epilogue = []
    for s in range(N_STRIPES):
        for j in range(3):
            g = _stage_slot(s, j)
            if g < TOT:
                slot_stages.setdefault(g, []).append((s, j))
            else:
                epilogue.append((g, s, j))
    epilogue.sort()

    barrier = pltpu.get_barrier_semaphore()
    for a in "xyz":
        pl.semaphore_signal(barrier, inc=1, device_id=nbr_of(a),
                            device_id_type=pl.DeviceIdType.MESH)
    pl.semaphore_wait(barrier, 3)

    pending = start_loads(0)
    for g in range(TOT):
        s, kt = divmod(g, NKT)
        c0, w, _ = STRIPES[s]
        for d in pending:
            d.wait()
        nxt = start_loads(g + 1) if g + 1 < TOT else ()
        pa = g % 2
        prod = lax.dot_general(
            a_buf[pa],
            b_buf[pa, pl.ds(0, w), :],
            (((1,), (1,)), ((), ())),
            preferred_element_type=jnp.float32,
        )
        sa = s % 2
        if kt == 0:
            acc[sa, :, pl.ds(0, w)] = prod
        else:
            acc[sa, :, pl.ds(0, w)] = acc[sa, :, pl.ds(0, w)] + prod
        if kt == NKT - 1:
            d = pltpu.make_async_remote_copy(
                src_ref=acc.at[sa, pl.ds((1 - x) * BLK, BLK), pl.ds(0, w)],
                dst_ref=rs_buf.at[s, :, pl.ds(0, w)],
                send_sem=rs_send.at[s],
                recv_sem=rs_recv.at[s],
                device_id=nbr_of("x"),
                device_id_type=pl.DeviceIdType.MESH,
            )
            d.start()
            rs_descs[s] = d
        for sj in slot_stages.get(g, ()):
            stage(*sj)
        pending = nxt

    for _, s, j in epilogue:
        stage(s, j)
    for s in range(N_STRIPES):
        for d in hop_descs[s]:
            d.wait()


def kernel(dy, W):
    return pl.pallas_call(
        _fused_body,
        out_shape=jax.ShapeDtypeStruct((M, N), jnp.float32),
        in_specs=[
            pl.BlockSpec(memory_space=pl.ANY),
            pl.BlockSpec(memory_space=pl.ANY),
        ],
        out_specs=pl.BlockSpec(memory_space=pl.ANY),
        scratch_shapes=[
            pltpu.VMEM((2, 2 * BLK, KT), jnp.float32),
            pltpu.VMEM((2, WMAX, KT), jnp.float32),
            pltpu.VMEM((2, 2 * BLK, WMAX), jnp.float32),
            pltpu.VMEM((N_STRIPES, BLK, WMAX), jnp.float32),
            pltpu.SemaphoreType.DMA((2, 2)),
            pltpu.SemaphoreType.DMA((2,)),
            pltpu.SemaphoreType.DMA((N_STRIPES,)),
            pltpu.SemaphoreType.DMA((N_STRIPES,)),
            pltpu.SemaphoreType.DMA((N_AG,)),
            pltpu.SemaphoreType.DMA((N_AG,)),
            pltpu.SemaphoreType.DMA((N_STRIPES,)),
        ],
        compiler_params=pltpu.CompilerParams(
            collective_id=0,
            vmem_limit_bytes=63 * 1024 * 1024,
        ),
    )(dy, W)


# device time: 478196 ns/iter; 1.0720x vs baseline; 1.0720x over previous
import jax
import jax.numpy as jnp
from jax import lax
from jax.experimental import pallas as pl
from jax.experimental.pallas import tpu as pltpu

M = 4096
N = 4096
K = 8192
BLK = M // 8

AXIS_W = {"x": 4, "y": 2, "z": 1}

STRIPES = (
    (0,    1024, ("z", "x", "y")),
    (1024, 1024, ("x", "y", "z")),
    (2048,  512, ("z", "x", "y")),
    (2560,  512, ("y", "x", "z")),
    (3072,  512, ("x", "y", "z")),
    (3584,  512, ("x", "z", "y")),
)
N_STRIPES = len(STRIPES)
WMAX = 1024
KT = 2048
NKT = K // KT
TOT = N_STRIPES * NKT

BRICKS = tuple(
    (s, STRIPES[s][0] + q * (STRIPES[s][1] // 2), STRIPES[s][1] // 2)
    for s in range(N_STRIPES)
    for q in range(2)
)
N_BRICKS = len(BRICKS)
WBMAX = 512

def _stage_slot(b, j):
    s, q = divmod(b, 2)
    return 4 * (s + 1) + 2 * q + 3 * j

N_AG = 68


def _runs(exchanged):
    offs = [0]
    for a in exchanged:
        w = AXIS_W[a]
        offs = sorted(o + d for o in offs for d in (0, w))
    runs = []
    start, length = offs[0], 1
    for o in offs[1:]:
        if o == start + length:
            length += 1
        else:
            runs.append((start, length))
            start, length = o, 1
    runs.append((start, length))
    return runs


def _fused_body(dy_ref, w_ref, out_ref, a_buf, b_buf, acc, rs_buf,
                a_sems, b_sems, rs_send, rs_recv, ag_send, ag_recv, cp_sems):
    x = lax.axis_index("x")
    y = lax.axis_index("y")
    z = lax.axis_index("z")
    idx = {"x": x, "y": y, "z": z}
    m = 2 * y + z
    b_own = 4 * x + m
    own_rows = pl.ds(b_own * BLK, BLK)

    def nbr_of(a):
        return tuple(1 - idx[ax] if ax == a else idx[ax] for ax in "xyz")

    def start_loads(g):
        s, kt = divmod(g, NKT)
        c0, w, _ = STRIPES[s]
        pa = g % 2
        descs = []
        for i, row0 in enumerate((m * BLK, (m + 4) * BLK)):
            d = pltpu.make_async_copy(
                dy_ref.at[pl.ds(row0, BLK), pl.ds(kt * KT, KT)],
                a_buf.at[pa, pl.ds(i * BLK, BLK), :],
                a_sems.at[pa, i],
            )
            d.start()
            descs.append(d)
        d = pltpu.make_async_copy(
            w_ref.at[pl.ds(c0, w), pl.ds(kt * KT, KT)],
            b_buf.at[pa, pl.ds(0, w), :],
            b_sems.at[pa],
        )
        d.start()
        descs.append(d)
        return descs

    ag_i = [0]
    exchanged = {}
    hop_descs = {}
    rs_descs = {}

    cp_descs = {}

    def start_hop(b):
        s, c0b, wb = BRICKS[b]
        order = STRIPES[s][2]
        ex = exchanged[b]
        a = order[len(ex)]
        base = sum(AXIS_W[ax] * idx[ax] for ax in "xyz" if ax not in ex)
        cols = pl.ds(c0b, wb)
        descs = []
        for off, length in _runs(ex):
            rows = pl.ds((base + off) * BLK, length * BLK)
            src = rs_buf.at[b, :, pl.ds(0, wb)] if not ex else out_ref.at[rows, cols]
            d = pltpu.make_async_remote_copy(
                src_ref=src,
                dst_ref=out_ref.at[rows, cols],
                send_sem=ag_send.at[ag_i[0]],
                recv_sem=ag_recv.at[ag_i[0]],
                device_id=nbr_of(a),
                device_id_type=pl.DeviceIdType.MESH,
            )
            d.start()
            descs.append(d)
            ag_i[0] += 1
        exchanged[b] = ex + (a,)
        hop_descs[b] = descs

    def stage(b, j):
        s, c0b, wb = BRICKS[b]
        coff = c0b - STRIPES[s][0]
        if j == 0:
            rs_descs[b].wait()
            sa = s % 2
            rs_buf[b, :, pl.ds(0, wb)] = (
                rs_buf[b, :, pl.ds(0, wb)]
                + acc[sa, pl.ds(x * BLK, BLK), pl.ds(coff, wb)]
            )
            cp = pltpu.make_async_copy(
                rs_buf.at[b, :, pl.ds(0, wb)],
                out_ref.at[own_rows, pl.ds(c0b, wb)],
                cp_sems.at[b],
            )
            cp.start()
            cp_descs[b] = cp
            exchanged[b] = ()
            start_hop(b)
        else:
            if j == 1:
                cp_descs[b].wait()
            for d in hop_descs[b]:
                d.wait()
            start_hop(b)

    slot_stages = {}
    epilogue = []
    for b in range(N_BRICKS):
        for j in range(3):
            g = _stage_slot(b, j)
            if g < TOT:
                slot_stages.setdefault(g, []).append((b, j))
            else:
                epilogue.append((g, b, j))
    epilogue.sort()

    barrier = pltpu.get_barrier_semaphore()
    for a in "xyz":
        pl.semaphore_signal(barrier, inc=1, device_id=nbr_of(a),
                            device_id_type=pl.DeviceIdType.MESH)
    pl.semaphore_wait(barrier, 3)

    pending = start_loads(0)
    for g in range(TOT):
        s, kt = divmod(g, NKT)
        c0, w, _ = STRIPES[s]
        for d in pending:
            d.wait()
        nxt = start_loads(g + 1) if g + 1 < TOT else ()
        pa = g % 2
        prod = lax.dot_general(
            a_buf[pa],
            b_buf[pa, pl.ds(0, w), :],
            (((1,), (1,)), ((), ())),
            preferred_element_type=jnp.float32,
        )
        sa = s % 2
        if kt == 0:
            acc[sa, :, pl.ds(0, w)] = prod
        else:
            acc[sa, :, pl.ds(0, w)] = acc[sa, :, pl.ds(0, w)] + prod
        if kt == NKT - 1:
            for b in (2 * s, 2 * s + 1):
                _, c0b, wb = BRICKS[b]
                coff = c0b - c0
                d = pltpu.make_async_remote_copy(
                    src_ref=acc.at[sa, pl.ds((1 - x) * BLK, BLK),
                                   pl.ds(coff, wb)],
                    dst_ref=rs_buf.at[b, :, pl.ds(0, wb)],
                    send_sem=rs_send.at[b],
                    recv_sem=rs_recv.at[b],
                    device_id=nbr_of("x"),
                    device_id_type=pl.DeviceIdType.MESH,
                )
                d.start()
                rs_descs[b] = d
        for bj in slot_stages.get(g, ()):
            stage(*bj)
        pending = nxt

    for _, b, j in epilogue:
        stage(b, j)
    for b in range(N_BRICKS):
        for d in hop_descs[b]:
            d.wait()


def kernel(dy, W):
    return pl.pallas_call(
        _fused_body,
        out_shape=jax.ShapeDtypeStruct((M, N), jnp.float32),
        in_specs=[
            pl.BlockSpec(memory_space=pl.ANY),
            pl.BlockSpec(memory_space=pl.ANY),
        ],
        out_specs=pl.BlockSpec(memory_space=pl.ANY),
        scratch_shapes=[
            pltpu.VMEM((2, 2 * BLK, KT), jnp.float32),
            pltpu.VMEM((2, WMAX, KT), jnp.float32),
            pltpu.VMEM((2, 2 * BLK, WMAX), jnp.float32),
            pltpu.VMEM((N_BRICKS, BLK, WBMAX), jnp.float32),
            pltpu.SemaphoreType.DMA((2, 2)),
            pltpu.SemaphoreType.DMA((2,)),
            pltpu.SemaphoreType.DMA((N_BRICKS,)),
            pltpu.SemaphoreType.DMA((N_BRICKS,)),
            pltpu.SemaphoreType.DMA((N_AG,)),
            pltpu.SemaphoreType.DMA((N_AG,)),
            pltpu.SemaphoreType.DMA((N_BRICKS,)),
        ],
        compiler_params=pltpu.CompilerParams(
            collective_id=0,
            vmem_limit_bytes=63 * 1024 * 1024,
        ),
    )(dy, W)


# device time: 451915 ns/iter; 1.1343x vs baseline; 1.0582x over previous
import jax
import jax.numpy as jnp
from jax import lax
from jax.experimental import pallas as pl
from jax.experimental.pallas import tpu as pltpu

M = 4096
N = 4096
K = 8192
BLK = M // 8

AXIS_W = {"x": 4, "y": 2, "z": 1}

STRIPES = (
    (0,    1408, ("z", "y", "x")),
    (1408, 1408, ("y", "x", "z")),
    (2816, 1280, ("x", "z", "y")),
)
N_STRIPES = len(STRIPES)
WMAX = 1408
KT = 1024
NKT = K // KT
TOT = N_STRIPES * NKT

BRICKS = tuple((s, STRIPES[s][0], STRIPES[s][1]) for s in range(N_STRIPES))
N_BRICKS = len(BRICKS)
WBMAX = 1408

def _stage_slot(b, j):
    return 8 * (b + 1) + 2 if j == 0 else 10 * TOT + b

N_AG = 15


def _runs(exchanged):
    offs = [0]
    for a in exchanged:
        w = AXIS_W[a]
        offs = sorted(o + d for o in offs for d in (0, w))
    runs = []
    start, length = offs[0], 1
    for o in offs[1:]:
        if o == start + length:
            length += 1
        else:
            runs.append((start, length))
            start, length = o, 1
    runs.append((start, length))
    return runs


def _fused_body(dy_ref, w_ref, out_ref, a_buf, b_buf, acc, rs_buf,
                a_sems, b_sems, rs_send, rs_recv, ag_send, ag_recv, cp_sems):
    x = lax.axis_index("x")
    y = lax.axis_index("y")
    z = lax.axis_index("z")
    idx = {"x": x, "y": y, "z": z}
    m = 2 * y + z
    b_own = 4 * x + m
    own_rows = pl.ds(b_own * BLK, BLK)

    def nbr_of(a):
        return tuple(1 - idx[ax] if ax == a else idx[ax] for ax in "xyz")

    def start_loads(g):
        s, kt = divmod(g, NKT)
        c0, w, _ = STRIPES[s]
        pa = g % 2
        descs = []
        for i, row0 in enumerate((m * BLK, (m + 4) * BLK)):
            d = pltpu.make_async_copy(
                dy_ref.at[pl.ds(row0, BLK), pl.ds(kt * KT, KT)],
                a_buf.at[pa, pl.ds(i * BLK, BLK), :],
                a_sems.at[pa, i],
            )
            d.start()
            descs.append(d)
        d = pltpu.make_async_copy(
            w_ref.at[pl.ds(c0, w), pl.ds(kt * KT, KT)],
            b_buf.at[pa, pl.ds(0, w), :],
            b_sems.at[pa],
        )
        d.start()
        descs.append(d)
        return descs

    ag_i = [0]
    exchanged = {}
    hop_descs = {}
    rs_descs = {}

    cp_descs = {}

    def start_hop(b):
        s, c0b, wb = BRICKS[b]
        order = STRIPES[s][2]
        ex = exchanged[b]
        a = order[len(ex)]
        base = sum(AXIS_W[ax] * idx[ax] for ax in "xyz" if ax not in ex)
        cols = pl.ds(c0b, wb)
        descs = []
        for off, length in _runs(ex):
            rows = pl.ds((base + off) * BLK, length * BLK)
            src = rs_buf.at[b, :, pl.ds(0, wb)] if not ex else out_ref.at[rows, cols]
            d = pltpu.make_async_remote_copy(
                src_ref=src,
                dst_ref=out_ref.at[rows, cols],
                send_sem=ag_send.at[ag_i[0]],
                recv_sem=ag_recv.at[ag_i[0]],
                device_id=nbr_of(a),
                device_id_type=pl.DeviceIdType.MESH,
            )
            d.start()
            descs.append(d)
            ag_i[0] += 1
        exchanged[b] = ex + (a,)
        hop_descs[b] = descs

    def stage(b, j):
        s, c0b, wb = BRICKS[b]
        coff = c0b - STRIPES[s][0]
        if j == 0:
            rs_descs[b].wait()
            sa = s % 2
            rs_buf[b, :, pl.ds(0, wb)] = (
                rs_buf[b, :, pl.ds(0, wb)]
                + acc[sa, pl.ds(x * BLK, BLK), pl.ds(coff, wb)]
            )
            cp = pltpu.make_async_copy(
                rs_buf.at[b, :, pl.ds(0, wb)],
                out_ref.at[own_rows, pl.ds(c0b, wb)],
                cp_sems.at[b],
            )
            cp.start()
            cp_descs[b] = cp
            exchanged[b] = ()
            start_hop(b)
        else:
            if j == 1:
                cp_descs[b].wait()
            for d in hop_descs[b]:
                d.wait()
            start_hop(b)

    slot_stages = {}
    epilogue = []
    for b in range(N_BRICKS):
        for j in range(3):
            g = _stage_slot(b, j)
            if g < TOT:
                slot_stages.setdefault(g, []).append((b, j))
            else:
                epilogue.append((g, b, j))
    epilogue.sort()

    barrier = pltpu.get_barrier_semaphore()
    for a in "xyz":
        pl.semaphore_signal(barrier, inc=1, device_id=nbr_of(a),
                            device_id_type=pl.DeviceIdType.MESH)
    pl.semaphore_wait(barrier, 3)

    pending = start_loads(0)
    for g in range(TOT):
        s, kt = divmod(g, NKT)
        c0, w, _ = STRIPES[s]
        for d in pending:
            d.wait()
        nxt = start_loads(g + 1) if g + 1 < TOT else ()
        pa = g % 2
        prod = lax.dot_general(
            a_buf[pa],
            b_buf[pa, pl.ds(0, w), :],
            (((1,), (1,)), ((), ())),
            preferred_element_type=jnp.float32,
        )
        sa = s % 2
        if kt == 0:
            acc[sa, :, pl.ds(0, w)] = prod
        else:
            acc[sa, :, pl.ds(0, w)] = acc[sa, :, pl.ds(0, w)] + prod
        if kt == NKT - 1:
            for b in [i for i, br in enumerate(BRICKS) if br[0] == s]:
                _, c0b, wb = BRICKS[b]
                coff = c0b - c0
                d = pltpu.make_async_remote_copy(
                    src_ref=acc.at[sa, pl.ds((1 - x) * BLK, BLK),
                                   pl.ds(coff, wb)],
                    dst_ref=rs_buf.at[b, :, pl.ds(0, wb)],
                    send_sem=rs_send.at[b],
                    recv_sem=rs_recv.at[b],
                    device_id=nbr_of("x"),
                    device_id_type=pl.DeviceIdType.MESH,
                )
                d.start()
                rs_descs[b] = d
        for bj in slot_stages.get(g, ()):
            stage(*bj)
        pending = nxt

    for _, b, j in epilogue:
        if j == 0:
            stage(b, j)
    for b in range(N_BRICKS):
        cp_descs[b].wait()
    for _hop in (1, 2):
        for b in range(N_BRICKS):
            for d in hop_descs[b]:
                d.wait()
        for b in range(N_BRICKS):
            start_hop(b)
    for b in range(N_BRICKS):
        for d in hop_descs[b]:
            d.wait()


def kernel(dy, W):
    return pl.pallas_call(
        _fused_body,
        out_shape=jax.ShapeDtypeStruct((M, N), jnp.float32),
        in_specs=[
            pl.BlockSpec(memory_space=pl.ANY),
            pl.BlockSpec(memory_space=pl.ANY),
        ],
        out_specs=pl.BlockSpec(memory_space=pl.ANY),
        scratch_shapes=[
            pltpu.VMEM((2, 2 * BLK, KT), jnp.float32),
            pltpu.VMEM((2, WMAX, KT), jnp.float32),
            pltpu.VMEM((2, 2 * BLK, WMAX), jnp.float32),
            pltpu.VMEM((N_BRICKS, BLK, WBMAX), jnp.float32),
            pltpu.SemaphoreType.DMA((2, 2)),
            pltpu.SemaphoreType.DMA((2,)),
            pltpu.SemaphoreType.DMA((N_BRICKS,)),
            pltpu.SemaphoreType.DMA((N_BRICKS,)),
            pltpu.SemaphoreType.DMA((N_AG,)),
            pltpu.SemaphoreType.DMA((N_AG,)),
            pltpu.SemaphoreType.DMA((N_BRICKS,)),
        ],
        compiler_params=pltpu.CompilerParams(
            collective_id=0,
            vmem_limit_bytes=63 * 1024 * 1024,
        ),
    )(dy, W)


# device time: 409774 ns/iter; 1.2510x vs baseline; 1.1028x over previous
import jax
import jax.numpy as jnp
from jax import lax
from jax.experimental import pallas as pl
from jax.experimental.pallas import tpu as pltpu

M = 4096
N = 4096
K = 8192
BLK = M // 8

AXIS_W = {"x": 4, "y": 2, "z": 1}

STRIPES = (
    (0,    1408, ("z", "y", "x")),
    (1408, 1408, ("y", "x", "z")),
    (2816, 1280, ("x", "z", "y")),
)
N_STRIPES = len(STRIPES)
WMAX = 1408
KT = 1024
NKT = K // KT
TOT = N_STRIPES * NKT

BRICKS = tuple((s, STRIPES[s][0], STRIPES[s][1]) for s in range(N_STRIPES))
N_BRICKS = len(BRICKS)
WBMAX = 1408

def _stage_slot(b, j):
    return (14, 22, 26)[b] if j == 0 else 10 * TOT + b

N_AG = 15


def _runs(exchanged):
    offs = [0]
    for a in exchanged:
        w = AXIS_W[a]
        offs = sorted(o + d for o in offs for d in (0, w))
    runs = []
    start, length = offs[0], 1
    for o in offs[1:]:
        if o == start + length:
            length += 1
        else:
            runs.append((start, length))
            start, length = o, 1
    runs.append((start, length))
    return runs


def _fused_body(dy_ref, w_ref, out_ref, a_buf, b_buf, acc, rs_buf,
                a_sems, b_sems, rs_send, rs_recv, ag_send, ag_recv, cp_sems):
    x = lax.axis_index("x")
    y = lax.axis_index("y")
    z = lax.axis_index("z")
    idx = {"x": x, "y": y, "z": z}
    m = 2 * y + z
    b_own = 4 * x + m
    own_rows = pl.ds(b_own * BLK, BLK)

    def nbr_of(a):
        return tuple(1 - idx[ax] if ax == a else idx[ax] for ax in "xyz")

    def start_loads(g):
        s, kt = divmod(g, NKT)
        c0, w, _ = STRIPES[s]
        pa = g % 2
        descs = []
        for i, row0 in enumerate((m * BLK, (m + 4) * BLK)):
            d = pltpu.make_async_copy(
                dy_ref.at[pl.ds(row0, BLK), pl.ds(kt * KT, KT)],
                a_buf.at[pa, pl.ds(i * BLK, BLK), :],
                a_sems.at[pa, i],
            )
            d.start()
            descs.append(d)
        d = pltpu.make_async_copy(
            w_ref.at[pl.ds(c0, w), pl.ds(kt * KT, KT)],
            b_buf.at[pa, pl.ds(0, w), :],
            b_sems.at[pa],
        )
        d.start()
        descs.append(d)
        return descs

    ag_i = [0]
    exchanged = {}
    hop_descs = {}
    rs_descs = {}

    cp_descs = {}

    def start_hop(b):
        s, c0b, wb = BRICKS[b]
        order = STRIPES[s][2]
        ex = exchanged[b]
        a = order[len(ex)]
        base = sum(AXIS_W[ax] * idx[ax] for ax in "xyz" if ax not in ex)
        cols = pl.ds(c0b, wb)
        descs = []
        for off, length in _runs(ex):
            rows = pl.ds((base + off) * BLK, length * BLK)
            src = rs_buf.at[b, :, pl.ds(0, wb)] if not ex else out_ref.at[rows, cols]
            d = pltpu.make_async_remote_copy(
                src_ref=src,
                dst_ref=out_ref.at[rows, cols],
                send_sem=ag_send.at[ag_i[0]],
                recv_sem=ag_recv.at[ag_i[0]],
                device_id=nbr_of(a),
                device_id_type=pl.DeviceIdType.MESH,
            )
            d.start()
            descs.append(d)
            ag_i[0] += 1
        exchanged[b] = ex + (a,)
        hop_descs[b] = descs

    def stage(b, j):
        s, c0b, wb = BRICKS[b]
        coff = c0b - STRIPES[s][0]
        if j == 0:
            rs_descs[b].wait()
            sa = s % 2
            rs_buf[b, :, pl.ds(0, wb)] = (
                rs_buf[b, :, pl.ds(0, wb)]
                + acc[sa, pl.ds(x * BLK, BLK), pl.ds(coff, wb)]
            )
            cp = pltpu.make_async_copy(
                rs_buf.at[b, :, pl.ds(0, wb)],
                out_ref.at[own_rows, pl.ds(c0b, wb)],
                cp_sems.at[b],
            )
            cp.start()
            cp_descs[b] = cp
            exchanged[b] = ()
            start_hop(b)
        else:
            if j == 1:
                cp_descs[b].wait()
            for d in hop_descs[b]:
                d.wait()
            start_hop(b)

    slot_stages = {}
    epilogue = []
    for b in range(N_BRICKS):
        for j in range(3):
            g = _stage_slot(b, j)
            if g < TOT:
                slot_stages.setdefault(g, []).append((b, j))
            else:
                epilogue.append((g, b, j))
    epilogue.sort()

    barrier = pltpu.get_barrier_semaphore()
    for a in "xyz":
        pl.semaphore_signal(barrier, inc=1, device_id=nbr_of(a),
                            device_id_type=pl.DeviceIdType.MESH)
    pl.semaphore_wait(barrier, 3)

    pending = start_loads(0)
    for g in range(TOT):
        s, kt = divmod(g, NKT)
        c0, w, _ = STRIPES[s]
        for d in pending:
            d.wait()
        nxt = start_loads(g + 1) if g + 1 < TOT else ()
        pa = g % 2
        prod = lax.dot_general(
            a_buf[pa],
            b_buf[pa, pl.ds(0, w), :],
            (((1,), (1,)), ((), ())),
            preferred_element_type=jnp.float32,
        )
        sa = s % 2
        if kt == 0:
            acc[sa, :, pl.ds(0, w)] = prod
        else:
            acc[sa, :, pl.ds(0, w)] = acc[sa, :, pl.ds(0, w)] + prod
        if kt == NKT - 1:
            for b in [i for i, br in enumerate(BRICKS) if br[0] == s]:
                _, c0b, wb = BRICKS[b]
                coff = c0b - c0
                d = pltpu.make_async_remote_copy(
                    src_ref=acc.at[sa, pl.ds((1 - x) * BLK, BLK),
                                   pl.ds(coff, wb)],
                    dst_ref=rs_buf.at[b, :, pl.ds(0, wb)],
                    send_sem=rs_send.at[b],
                    recv_sem=rs_recv.at[b],
                    device_id=nbr_of("x"),
                    device_id_type=pl.DeviceIdType.MESH,
                )
                d.start()
                rs_descs[b] = d
        for bj in slot_stages.get(g, ()):
            stage(*bj)
        pending = nxt

    for _, b, j in epilogue:
        if j == 0:
            stage(b, j)
    for b in range(N_BRICKS):
        cp_descs[b].wait()
    for _hop in (1, 2):
        for b in range(N_BRICKS):
            for d in hop_descs[b]:
                d.wait()
        for b in range(N_BRICKS):
            start_hop(b)
    for b in range(N_BRICKS):
        for d in hop_descs[b]:
            d.wait()


def kernel(dy, W):
    return pl.pallas_call(
        _fused_body,
        out_shape=jax.ShapeDtypeStruct((M, N), jnp.float32),
        in_specs=[
            pl.BlockSpec(memory_space=pl.ANY),
            pl.BlockSpec(memory_space=pl.ANY),
        ],
        out_specs=pl.BlockSpec(memory_space=pl.ANY),
        scratch_shapes=[
            pltpu.VMEM((2, 2 * BLK, KT), jnp.float32),
            pltpu.VMEM((2, WMAX, KT), jnp.float32),
            pltpu.VMEM((2, 2 * BLK, WMAX), jnp.float32),
            pltpu.VMEM((N_BRICKS, BLK, WBMAX), jnp.float32),
            pltpu.SemaphoreType.DMA((2, 2)),
            pltpu.SemaphoreType.DMA((2,)),
            pltpu.SemaphoreType.DMA((N_BRICKS,)),
            pltpu.SemaphoreType.DMA((N_BRICKS,)),
            pltpu.SemaphoreType.DMA((N_AG,)),
            pltpu.SemaphoreType.DMA((N_AG,)),
            pltpu.SemaphoreType.DMA((N_BRICKS,)),
        ],
        compiler_params=pltpu.CompilerParams(
            collective_id=0,
            vmem_limit_bytes=63 * 1024 * 1024,
        ),
    )(dy, W)
